# baseline (speedup 1.0000x reference)
import functools

import jax
import jax.numpy as jnp
from jax import lax
from jax.experimental import pallas as pl
from jax.experimental.pallas import tpu as pltpu
from jax.experimental.pallas import tpu_sc as plsc


_VB = 1024


def _proj_body(tt_ref, w_ref, b_ref, a_ref, out_ref):
    z = jnp.dot(
        w_ref[...], tt_ref[...],
        preferred_element_type=jnp.float32,
    )
    z = z + b_ref[:, 0:1]
    zt = z.T
    sel = jnp.dot(
        a_ref[...], zt,
        preferred_element_type=jnp.float32,
        precision=lax.Precision.HIGHEST,
    )
    s = zt.shape[1]
    g = 128 // s
    pieces = [sel[j * _VB // g:(j + 1) * _VB // g, :] for j in range(g)]
    out_ref[...] = jnp.concatenate(pieces, axis=1)


def _pack_perm(vb: int, s: int):
    import numpy as np
    g = 128 // s
    rows = vb // g
    a = np.zeros((vb, vb), np.float32)
    for j in range(g):
        for q in range(rows):
            a[j * rows + q, g * q + j] = 1.0
    return jnp.asarray(a)


@functools.cache
def _make_proj(vocab: int, d: int, s: int):
    nb = pl.cdiv(vocab, _VB)
    return pl.pallas_call(
        _proj_body,
        grid=(nb,),
        in_specs=[
            pl.BlockSpec((d, _VB), lambda i: (0, i)),
            pl.BlockSpec((s, d), lambda i: (0, 0)),
            pl.BlockSpec((s, 128), lambda i: (0, 0)),
            pl.BlockSpec((_VB, _VB), lambda i: (0, 0)),
        ],
        out_specs=pl.BlockSpec((_VB * s // 128, 128), lambda i: (i, 0)),
        out_shape=jax.ShapeDtypeStruct((vocab * s // 128, 128), jnp.float32),
    )


_IDX_MINOR = 128
_ROWS_PER_STEP = 8


@functools.cache
def _make_sc_gather(n_rows: int, d: int, vocab: int):
    info = plsc.get_sparse_core_info()
    nc, ns = info.num_cores, info.num_subcores
    nw = nc * ns
    rows_per_w = n_rows // nw
    assert rows_per_w * nw == n_rows
    r = _ROWS_PER_STEP
    n_steps = rows_per_w // r
    assert n_steps * r == rows_per_w and n_steps >= 2
    mesh = plsc.VectorSubcoreMesh(core_axis_name="c", subcore_axis_name="s")

    @functools.partial(
        pl.kernel,
        mesh=mesh,
        compiler_params=pltpu.CompilerParams(use_tc_tiling_on_sc=False),
        out_type=jax.ShapeDtypeStruct((n_rows, _IDX_MINOR, d), jnp.float32),
        scratch_types=[
            pltpu.VMEM((2, r, _IDX_MINOR), jnp.int32),
            pltpu.VMEM((2, r, _IDX_MINOR, d), jnp.float32),
            pltpu.SemaphoreType.DMA,
            pltpu.SemaphoreType.DMA,
        ],
    )
    def sc_gather(idx_hbm, table_hbm, out_hbm, idx_v, rows_v, gsem, wsem):
        wid = lax.axis_index("s") * nc + lax.axis_index("c")
        row0 = wid * rows_per_w

        def fire(o, buf):
            base = row0 + o * r
            pltpu.sync_copy(idx_hbm.at[pl.ds(base, r)], idx_v.at[buf])
            for j in range(r):
                pltpu.async_copy(
                    table_hbm.at[idx_v.at[buf, j]], rows_v.at[buf, j], gsem
                )

        def drain_gathers(buf):
            for j in range(r):
                pltpu.make_async_copy(
                    table_hbm.at[idx_v.at[buf, j]], rows_v.at[buf, j], gsem
                ).wait()

        def writeback(o, buf):
            base = row0 + o * r
            return pltpu.async_copy(
                rows_v.at[buf], out_hbm.at[pl.ds(base, r)], wsem
            )

        def wait_writeback(o, buf):
            base = row0 + o * r
            pltpu.make_async_copy(
                rows_v.at[buf], out_hbm.at[pl.ds(base, r)], wsem
            ).wait()

        fire(0, 0)

        def step(o, carry):
            buf = lax.rem(o, 2)
            nxt = lax.rem(o + 1, 2)

            @pl.when(o > 0)
            def _():
                wait_writeback(o - 1, nxt)

            @pl.when(o + 1 < n_steps)
            def _():
                fire(o + 1, nxt)

            drain_gathers(buf)
            writeback(o, buf)
            return carry

        lax.fori_loop(0, n_steps, step, 0)
        wait_writeback(n_steps - 1, lax.rem(n_steps - 1, 2))

    return sc_gather


_TB = 1024


def _group_butterfly(x, lane, op):
    for k in (1, 2, 4, 8):
        fwd = pltpu.roll(x, 128 - k, 1)
        bwd = pltpu.roll(x, k, 1)
        x = op(x, jnp.where((lane & k) == 0, fwd, bwd))
    return x


def _argmax_body(z_ref, g_ref, out_ref):
    x = z_ref[...] + g_ref[...]
    lane = lax.broadcasted_iota(jnp.int32, x.shape, 1)
    m = _group_butterfly(x, lane, jnp.maximum)
    ii = lane & 15
    cand = jnp.where(x == m, ii, 16)
    first = _group_butterfly(cand, lane, jnp.minimum)
    out_ref[...] = (ii == first).astype(jnp.float32)


@functools.cache
def _make_argmax(rows: int):
    assert rows % _TB == 0
    return pl.pallas_call(
        _argmax_body,
        grid=(rows // _TB,),
        in_specs=[
            pl.BlockSpec((_TB, 128), lambda i: (i, 0)),
            pl.BlockSpec((_TB, 128), lambda i: (i, 0)),
        ],
        out_specs=pl.BlockSpec((_TB, 128), lambda i: (i, 0)),
        out_shape=jax.ShapeDtypeStruct((rows, 128), jnp.float32),
    )


def kernel(input_ids, emb_table, W, b):
    bb, ll = input_ids.shape
    vocab, d = emb_table.shape
    s = W.shape[0]
    n = bb * ll

    b128 = jnp.broadcast_to(b.reshape(s, 1), (s, 128))
    with jax.ensure_compile_time_eval():
        perm = _pack_perm(_VB, s)
    p8 = _make_proj(vocab, d, s)(emb_table.T, W, b128, perm)
    p_rows = p8.reshape(vocab, s)

    ids = input_ids.reshape(n // _IDX_MINOR, _IDX_MINOR).astype(jnp.int32)
    z3 = _make_sc_gather(n // _IDX_MINOR, s, vocab)(ids, p_rows)
    z8 = z3.reshape(n * s // 128, 128)

    with jax.ensure_compile_time_eval():
        u = jax.random.uniform(
            jax.random.key(42), (bb, ll, s), jnp.float32, 1e-10, 1.0
        )
        g8 = (-jnp.log(-jnp.log(u))).reshape(n * s // 128, 128)

    out8 = _make_argmax(n * s // 128)(z8, g8)
    return out8.reshape(bb, ll, s)


# speedup vs baseline: 3.0127x; 3.0127x over previous
import functools

import jax
import jax.numpy as jnp
from jax import lax
from jax.experimental import pallas as pl
from jax.experimental.pallas import tpu as pltpu
from jax.experimental.pallas import tpu_sc as plsc


_VB = 8192


def _proj_body(tt_ref, w_ref, b_ref, out_ref):
    z = jnp.dot(
        w_ref[...], tt_ref[...],
        preferred_element_type=jnp.float32,
    )
    out_ref[...] = z + b_ref[:, 0:1]


@functools.cache
def _make_proj(vocab: int, d: int, s: int):
    nb = pl.cdiv(vocab, _VB)
    return pl.pallas_call(
        _proj_body,
        grid=(nb,),
        in_specs=[
            pl.BlockSpec((d, _VB), lambda i: (0, i)),
            pl.BlockSpec((s, d), lambda i: (0, 0)),
            pl.BlockSpec((s, 128), lambda i: (0, 0)),
        ],
        out_specs=pl.BlockSpec((s, _VB), lambda i: (0, i)),
        out_shape=jax.ShapeDtypeStruct((s, vocab), jnp.float32),
    )


_IDX_MINOR = 128
_ROWS_PER_STEP = 8


@functools.cache
def _make_sc_gather(n_rows: int, d: int, vocab: int):
    info = plsc.get_sparse_core_info()
    nc, ns = info.num_cores, info.num_subcores
    nw = nc * ns
    rows_per_w = n_rows // nw
    assert rows_per_w * nw == n_rows
    r = _ROWS_PER_STEP
    n_steps = rows_per_w // r
    assert n_steps * r == rows_per_w and n_steps >= 2
    mesh = plsc.VectorSubcoreMesh(core_axis_name="c", subcore_axis_name="s")

    @functools.partial(
        pl.kernel,
        mesh=mesh,
        compiler_params=pltpu.CompilerParams(use_tc_tiling_on_sc=False),
        out_type=jax.ShapeDtypeStruct((n_rows, _IDX_MINOR, d), jnp.float32),
        scratch_types=[
            pltpu.VMEM((2, r, _IDX_MINOR), jnp.int32),
            pltpu.VMEM((2, r, _IDX_MINOR, d), jnp.float32),
            pltpu.SemaphoreType.DMA,
            pltpu.SemaphoreType.DMA,
        ],
    )
    def sc_gather(idx_hbm, table_hbm, out_hbm, idx_v, rows_v, gsem, wsem):
        wid = lax.axis_index("s") * nc + lax.axis_index("c")
        row0 = wid * rows_per_w

        def fire(o, buf):
            base = row0 + o * r
            pltpu.sync_copy(idx_hbm.at[pl.ds(base, r)], idx_v.at[buf])
            for j in range(r):
                pltpu.async_copy(
                    table_hbm.at[idx_v.at[buf, j]], rows_v.at[buf, j], gsem
                )

        def drain_gathers(buf):
            for j in range(r):
                pltpu.make_async_copy(
                    table_hbm.at[idx_v.at[buf, j]], rows_v.at[buf, j], gsem
                ).wait()

        def writeback(o, buf):
            base = row0 + o * r
            return pltpu.async_copy(
                rows_v.at[buf], out_hbm.at[pl.ds(base, r)], wsem
            )

        def wait_writeback(o, buf):
            base = row0 + o * r
            pltpu.make_async_copy(
                rows_v.at[buf], out_hbm.at[pl.ds(base, r)], wsem
            ).wait()

        fire(0, 0)

        def step(o, carry):
            buf = lax.rem(o, 2)
            nxt = lax.rem(o + 1, 2)

            @pl.when(o > 0)
            def _():
                wait_writeback(o - 1, nxt)

            @pl.when(o + 1 < n_steps)
            def _():
                fire(o + 1, nxt)

            drain_gathers(buf)
            writeback(o, buf)
            return carry

        lax.fori_loop(0, n_steps, step, 0)
        wait_writeback(n_steps - 1, lax.rem(n_steps - 1, 2))

    return sc_gather


_TB = 1024


def _group_butterfly(x, lane, op):
    for k in (1, 2, 4, 8):
        fwd = pltpu.roll(x, 128 - k, 1)
        bwd = pltpu.roll(x, k, 1)
        x = op(x, jnp.where((lane & k) == 0, fwd, bwd))
    return x


def _argmax_body(z_ref, g_ref, out_ref):
    x = z_ref[...] + g_ref[...]
    lane = lax.broadcasted_iota(jnp.int32, x.shape, 1)
    m = _group_butterfly(x, lane, jnp.maximum)
    ii = lane & 15
    cand = jnp.where(x == m, ii, 16)
    first = _group_butterfly(cand, lane, jnp.minimum)
    out_ref[...] = (ii == first).astype(jnp.float32)


@functools.cache
def _make_argmax(rows: int):
    assert rows % _TB == 0
    return pl.pallas_call(
        _argmax_body,
        grid=(rows // _TB,),
        in_specs=[
            pl.BlockSpec((_TB, 128), lambda i: (i, 0)),
            pl.BlockSpec((_TB, 128), lambda i: (i, 0)),
        ],
        out_specs=pl.BlockSpec((_TB, 128), lambda i: (i, 0)),
        out_shape=jax.ShapeDtypeStruct((rows, 128), jnp.float32),
    )


def kernel(input_ids, emb_table, W, b):
    bb, ll = input_ids.shape
    vocab, d = emb_table.shape
    s = W.shape[0]
    n = bb * ll

    b128 = jnp.broadcast_to(b.reshape(s, 1), (s, 128))
    pt = _make_proj(vocab, d, s)(emb_table.T, W, b128)
    p_rows = pt.T

    ids = input_ids.reshape(n // _IDX_MINOR, _IDX_MINOR).astype(jnp.int32)
    z3 = _make_sc_gather(n // _IDX_MINOR, s, vocab)(ids, p_rows)
    z8 = z3.reshape(n * s // 128, 128)

    with jax.ensure_compile_time_eval():
        u = jax.random.uniform(
            jax.random.key(42), (bb, ll, s), jnp.float32, 1e-10, 1.0
        )
        g8 = (-jnp.log(-jnp.log(u))).reshape(n * s // 128, 128)

    out8 = _make_argmax(n * s // 128)(z8, g8)
    return out8.reshape(bb, ll, s)


# speedup vs baseline: 4.0899x; 1.3575x over previous
import functools

import jax
import jax.numpy as jnp
from jax import lax
from jax.experimental import pallas as pl
from jax.experimental.pallas import tpu as pltpu
from jax.experimental.pallas import tpu_sc as plsc


_VB = 8192


def _proj_body(tt_ref, w_ref, b_ref, out_ref):
    z = jnp.dot(
        w_ref[...], tt_ref[...],
        preferred_element_type=jnp.float32,
    )
    z = z + b_ref[:, 0:1]
    s = z.shape[0]
    g = 128 // s
    zt = z.T
    zt3 = zt.reshape(_VB // g, g, s)
    t3 = jnp.transpose(zt3, (1, 0, 2))
    out_ref[...] = jnp.concatenate([t3[j] for j in range(g)], axis=1)


@functools.cache
def _make_proj(vocab: int, d: int, s: int):
    nb = pl.cdiv(vocab, _VB)
    return pl.pallas_call(
        _proj_body,
        grid=(nb,),
        in_specs=[
            pl.BlockSpec((d, _VB), lambda i: (0, i)),
            pl.BlockSpec((s, d), lambda i: (0, 0)),
            pl.BlockSpec((s, 128), lambda i: (0, 0)),
        ],
        out_specs=pl.BlockSpec((_VB * s // 128, 128), lambda i: (i, 0)),
        out_shape=jax.ShapeDtypeStruct((vocab * s // 128, 128), jnp.float32),
    )


_IDX_MINOR = 128
_ROWS_PER_STEP = 8


@functools.cache
def _make_sc_gather(n_rows: int, d: int, vocab: int):
    info = plsc.get_sparse_core_info()
    nc, ns = info.num_cores, info.num_subcores
    nw = nc * ns
    rows_per_w = n_rows // nw
    assert rows_per_w * nw == n_rows
    r = _ROWS_PER_STEP
    n_steps = rows_per_w // r
    assert n_steps * r == rows_per_w and n_steps >= 2
    mesh = plsc.VectorSubcoreMesh(core_axis_name="c", subcore_axis_name="s")

    @functools.partial(
        pl.kernel,
        mesh=mesh,
        compiler_params=pltpu.CompilerParams(use_tc_tiling_on_sc=False),
        out_type=jax.ShapeDtypeStruct((n_rows, _IDX_MINOR, d), jnp.float32),
        scratch_types=[
            pltpu.VMEM((2, r, _IDX_MINOR), jnp.int32),
            pltpu.VMEM((2, r, _IDX_MINOR, d), jnp.float32),
            pltpu.SemaphoreType.DMA,
            pltpu.SemaphoreType.DMA,
        ],
    )
    def sc_gather(idx_hbm, table_hbm, out_hbm, idx_v, rows_v, gsem, wsem):
        wid = lax.axis_index("s") * nc + lax.axis_index("c")
        row0 = wid * rows_per_w

        def fire(o, buf):
            base = row0 + o * r
            pltpu.sync_copy(idx_hbm.at[pl.ds(base, r)], idx_v.at[buf])
            for j in range(r):
                pltpu.async_copy(
                    table_hbm.at[idx_v.at[buf, j]], rows_v.at[buf, j], gsem
                )

        def drain_gathers(buf):
            for j in range(r):
                pltpu.make_async_copy(
                    table_hbm.at[idx_v.at[buf, j]], rows_v.at[buf, j], gsem
                ).wait()

        def writeback(o, buf):
            base = row0 + o * r
            return pltpu.async_copy(
                rows_v.at[buf], out_hbm.at[pl.ds(base, r)], wsem
            )

        def wait_writeback(o, buf):
            base = row0 + o * r
            pltpu.make_async_copy(
                rows_v.at[buf], out_hbm.at[pl.ds(base, r)], wsem
            ).wait()

        fire(0, 0)

        def step(o, carry):
            buf = lax.rem(o, 2)
            nxt = lax.rem(o + 1, 2)

            @pl.when(o > 0)
            def _():
                wait_writeback(o - 1, nxt)

            @pl.when(o + 1 < n_steps)
            def _():
                fire(o + 1, nxt)

            drain_gathers(buf)
            writeback(o, buf)
            return carry

        lax.fori_loop(0, n_steps, step, 0)
        wait_writeback(n_steps - 1, lax.rem(n_steps - 1, 2))

    return sc_gather


_TB = 1024


def _group_butterfly(x, lane, op):
    for k in (1, 2, 4, 8):
        fwd = pltpu.roll(x, 128 - k, 1)
        bwd = pltpu.roll(x, k, 1)
        x = op(x, jnp.where((lane & k) == 0, fwd, bwd))
    return x


def _argmax_body(z_ref, g_ref, out_ref):
    x = z_ref[...] + g_ref[...]
    lane = lax.broadcasted_iota(jnp.int32, x.shape, 1)
    m = _group_butterfly(x, lane, jnp.maximum)
    ii = lane & 15
    cand = jnp.where(x == m, ii, 16)
    first = _group_butterfly(cand, lane, jnp.minimum)
    out_ref[...] = (ii == first).astype(jnp.float32)


@functools.cache
def _make_argmax(rows: int):
    assert rows % _TB == 0
    return pl.pallas_call(
        _argmax_body,
        grid=(rows // _TB,),
        in_specs=[
            pl.BlockSpec((_TB, 128), lambda i: (i, 0)),
            pl.BlockSpec((_TB, 128), lambda i: (i, 0)),
        ],
        out_specs=pl.BlockSpec((_TB, 128), lambda i: (i, 0)),
        out_shape=jax.ShapeDtypeStruct((rows, 128), jnp.float32),
    )


def kernel(input_ids, emb_table, W, b):
    bb, ll = input_ids.shape
    vocab, d = emb_table.shape
    s = W.shape[0]
    n = bb * ll

    b128 = jnp.broadcast_to(b.reshape(s, 1), (s, 128))
    p8 = _make_proj(vocab, d, s)(emb_table.T, W, b128)
    p_rows = p8.reshape(vocab, s)

    ids = input_ids.reshape(n // _IDX_MINOR, _IDX_MINOR).astype(jnp.int32)
    z3 = _make_sc_gather(n // _IDX_MINOR, s, vocab)(ids, p_rows)
    z8 = z3.reshape(n * s // 128, 128)

    with jax.ensure_compile_time_eval():
        u = jax.random.uniform(
            jax.random.key(42), (bb, ll, s), jnp.float32, 1e-10, 1.0
        )
        g8 = (-jnp.log(-jnp.log(u))).reshape(n * s // 128, 128)

    out8 = _make_argmax(n * s // 128)(z8, g8)
    return out8.reshape(bb, ll, s)


# speedup vs baseline: 4.6662x; 1.1409x over previous
import functools

import jax
import jax.numpy as jnp
from jax import lax
from jax.experimental import pallas as pl
from jax.experimental.pallas import tpu as pltpu
from jax.experimental.pallas import tpu_sc as plsc


_VB = 8192


def _proj_body(tt_ref, w_ref, b_ref, out_ref):
    z = jnp.dot(
        w_ref[...], tt_ref[...],
        preferred_element_type=jnp.float32,
    )
    z = z + b_ref[:, 0:1]
    s = z.shape[0]
    g = 128 // s
    zt = z.T
    zt3 = zt.reshape(_VB // g, g, s)
    t3 = jnp.transpose(zt3, (1, 0, 2))
    out_ref[...] = jnp.concatenate([t3[j] for j in range(g)], axis=1)


@functools.cache
def _make_proj(vocab: int, d: int, s: int):
    nb = pl.cdiv(vocab, _VB)
    return pl.pallas_call(
        _proj_body,
        grid=(nb,),
        in_specs=[
            pl.BlockSpec((d, _VB), lambda i: (0, i)),
            pl.BlockSpec((s, d), lambda i: (0, 0)),
            pl.BlockSpec((s, 128), lambda i: (0, 0)),
        ],
        out_specs=pl.BlockSpec((_VB * s // 128, 128), lambda i: (i, 0)),
        out_shape=jax.ShapeDtypeStruct((vocab * s // 128, 128), jnp.float32),
    )


_IDX_MINOR = 128
_ROWS_PER_STEP = 8


@functools.cache
def _make_sc_gather(n_rows: int, d: int, vocab: int):
    info = plsc.get_sparse_core_info()
    nc, ns = info.num_cores, info.num_subcores
    nw = nc * ns
    rows_per_w = n_rows // nw
    assert rows_per_w * nw == n_rows
    r = _ROWS_PER_STEP
    n_steps = rows_per_w // r
    assert n_steps * r == rows_per_w and n_steps >= 2
    mesh = plsc.VectorSubcoreMesh(core_axis_name="c", subcore_axis_name="s")

    @functools.partial(
        pl.kernel,
        mesh=mesh,
        compiler_params=pltpu.CompilerParams(use_tc_tiling_on_sc=False),
        out_type=jax.ShapeDtypeStruct((n_rows, _IDX_MINOR, d), jnp.float32),
        scratch_types=[
            pltpu.VMEM((2, r, _IDX_MINOR), jnp.int32),
            pltpu.VMEM((2, r, _IDX_MINOR, d), jnp.float32),
            pltpu.SemaphoreType.DMA,
            pltpu.SemaphoreType.DMA,
        ],
    )
    def sc_gather(idx_hbm, table_hbm, out_hbm, idx_v, rows_v, gsem, wsem):
        wid = lax.axis_index("s") * nc + lax.axis_index("c")
        row0 = wid * rows_per_w

        def fire(o, buf):
            base = row0 + o * r
            pltpu.sync_copy(idx_hbm.at[pl.ds(base, r)], idx_v.at[buf])
            for j in range(r):
                pltpu.async_copy(
                    table_hbm.at[idx_v.at[buf, j]], rows_v.at[buf, j], gsem
                )

        def drain_gathers(buf):
            for j in range(r):
                pltpu.make_async_copy(
                    table_hbm.at[idx_v.at[buf, j]], rows_v.at[buf, j], gsem
                ).wait()

        def writeback(o, buf):
            base = row0 + o * r
            return pltpu.async_copy(
                rows_v.at[buf], out_hbm.at[pl.ds(base, r)], wsem
            )

        def wait_writeback(o, buf):
            base = row0 + o * r
            pltpu.make_async_copy(
                rows_v.at[buf], out_hbm.at[pl.ds(base, r)], wsem
            ).wait()

        fire(0, 0)

        def step(o, carry):
            buf = lax.rem(o, 2)
            nxt = lax.rem(o + 1, 2)

            @pl.when(o > 0)
            def _():
                wait_writeback(o - 1, nxt)

            @pl.when(o + 1 < n_steps)
            def _():
                fire(o + 1, nxt)

            drain_gathers(buf)
            writeback(o, buf)
            return carry

        lax.fori_loop(0, n_steps, step, 0)
        wait_writeback(n_steps - 1, lax.rem(n_steps - 1, 2))

    return sc_gather


_TB = 1024


def _group_butterfly(x, lane, op):
    for k in (1, 2, 4, 8):
        fwd = pltpu.roll(x, 128 - k, 1)
        bwd = pltpu.roll(x, k, 1)
        x = op(x, jnp.where((lane & k) == 0, fwd, bwd))
    return x


def _argmax_body(z_ref, g_ref, out_ref):
    x = z_ref[...] + g_ref[...]
    lane = lax.broadcasted_iota(jnp.int32, x.shape, 1)
    m = _group_butterfly(x, lane, jnp.maximum)
    ii = lane & 15
    cand = jnp.where(x == m, ii, 16)
    first = _group_butterfly(cand, lane, jnp.minimum)
    out_ref[...] = (ii == first).astype(jnp.float32)


@functools.cache
def _make_argmax(rows: int):
    assert rows % _TB == 0
    return pl.pallas_call(
        _argmax_body,
        grid=(rows // _TB,),
        in_specs=[
            pl.BlockSpec((_TB, 128), lambda i: (i, 0)),
            pl.BlockSpec((_TB, 128), lambda i: (i, 0)),
        ],
        out_specs=pl.BlockSpec((_TB, 128), lambda i: (i, 0)),
        out_shape=jax.ShapeDtypeStruct((rows, 128), jnp.float32),
    )


def kernel(input_ids, emb_table, W, b):
    bb, ll = input_ids.shape
    vocab, d = emb_table.shape
    s = W.shape[0]
    n = bb * ll

    b128 = jnp.broadcast_to(b.reshape(s, 1), (s, 128))
    p8 = _make_proj(vocab, d, s)(emb_table.T, W, b128)
    p_rows = p8.reshape(vocab, s)

    ids = input_ids.T.reshape(n // _IDX_MINOR, _IDX_MINOR).astype(jnp.int32)
    z3 = _make_sc_gather(n // _IDX_MINOR, s, vocab)(ids, p_rows)
    z8 = z3.reshape(n * s // 128, 128)

    with jax.ensure_compile_time_eval():
        u = jax.random.uniform(
            jax.random.key(42), (bb, ll, s), jnp.float32, 1e-10, 1.0
        )
        g = -jnp.log(-jnp.log(u))
        g8 = jnp.transpose(g, (1, 0, 2)).reshape(n * s // 128, 128)

    out8 = _make_argmax(n * s // 128)(z8, g8)
    return jnp.transpose(out8.reshape(ll, bb, s), (1, 0, 2))
